# baseline (speedup 1.0000x reference)
import functools

import jax
import jax.numpy as jnp
from jax import lax
from jax.experimental import pallas as pl
from jax.experimental.pallas import tpu as pltpu
from jax.experimental.pallas import tpu_sc as plsc

NUM_NODES = 1000000
DIM = 64
BATCH = 16384

_INFO = plsc.get_sparse_core_info()
_NC = _INFO.num_cores
_NS = _INFO.num_subcores
_NW = _NC * _NS
_BPW = BATCH // _NW
_NCHUNK = 4
_CROWS = _BPW // _NCHUNK
_NGRP = _CROWS // 16


def _log_sigmoid(x):
    t = jnp.exp(-jnp.abs(x))
    z = t / (t + 2.0)
    z2 = z * z
    log1p = 2.0 * z * (1.0 + z2 * (1.0 / 3.0 + z2 * (0.2 + z2 * (1.0 / 7.0 + z2 * (1.0 / 9.0)))))
    return jnp.minimum(x, 0.0) - log1p


@functools.partial(
    pl.kernel,
    out_type=jax.ShapeDtypeStruct((_NW, 128), jnp.float32),
    mesh=plsc.VectorSubcoreMesh(core_axis_name="c", subcore_axis_name="s"),
    compiler_params=pltpu.CompilerParams(
        needs_layout_passes=False, use_tc_tiling_on_sc=True),
    scratch_types=[
        pltpu.VMEM((_NCHUNK, _CROWS), jnp.int32),
        pltpu.VMEM((_NCHUNK, _CROWS), jnp.int32),
        pltpu.VMEM((_NCHUNK, _CROWS), jnp.int32),
        pltpu.VMEM((_NCHUNK, _CROWS), jnp.int32),
        pltpu.VMEM((_CROWS, 128), jnp.float32),
        pltpu.VMEM((_CROWS, 128), jnp.float32),
        pltpu.VMEM((_CROWS, 128), jnp.float32),
        pltpu.VMEM((_CROWS, 128), jnp.float32),
        pltpu.VMEM((_BPW,), jnp.float32),
        pltpu.VMEM((128,), jnp.float32),
        pltpu.SemaphoreType.DMA,
        pltpu.SemaphoreType.DMA,
    ],
)
def _sc_loss(u_hbm, v_hbm, sign_hbm, emb_hbm, ctx_hbm, out_hbm,
             u_idx, v_idx, up_idx, vp_idx, ebuf0, ebuf1, cbuf0, cbuf1,
             sign_v, stage_v, sem0, sem1):
    wid = lax.axis_index("s") * _NC + lax.axis_index("c")
    base = wid * _BPW

    for j in range(_NCHUNK):
        pltpu.sync_copy(u_hbm.at[pl.ds(base + j * _CROWS, _CROWS)], u_idx.at[j])
        pltpu.sync_copy(v_hbm.at[pl.ds(base + j * _CROWS, _CROWS)], v_idx.at[j])
    pltpu.sync_copy(sign_hbm.at[pl.ds(base, _BPW)], sign_v)

    for j in range(_NCHUNK):
        for k in range(_CROWS // 16):
            sl = pl.ds(k * 16, 16)
            up_idx[j, sl] = lax.shift_right_logical(u_idx[j, sl], 1)
            vp_idx[j, sl] = lax.shift_right_logical(v_idx[j, sl], 1)

    bufs = [(ebuf0, cbuf0), (ebuf1, cbuf1)]
    sems = [sem0, sem1]
    handles = [None] * _NCHUNK

    def fire(c):
        eb, cb = bufs[c % 2]
        handles[c] = (
            pltpu.async_copy(emb_hbm.at[up_idx.at[c]], eb, sems[c % 2]),
            pltpu.async_copy(ctx_hbm.at[vp_idx.at[c]], cb, sems[c % 2]),
        )

    fire(0)
    fire(1)

    lane = lax.iota(jnp.int32, 16)
    loss = jnp.zeros((16,), jnp.float32)
    for c in range(_NCHUNK):
        eb, cb = bufs[c % 2]
        he, hc = handles[c]
        he.wait()
        hc.wait()

        def group_body(g, acc_loss, c=c, eb=eb, cb=cb):
            rows = g * 16 + lane
            gsl = pl.ds(g * 16, 16)
            ucol = (u_idx[c, gsl] & 1) * DIM
            vcol = (v_idx[c, gsl] & 1) * DIM
            acc = jnp.zeros((16,), jnp.float32)
            for col in range(DIM):
                e = plsc.load_gather(eb, [rows, ucol + col])
                x = plsc.load_gather(cb, [rows, vcol + col])
                acc = acc + e * x
            x = acc * sign_v[pl.ds(c * _CROWS + g * 16, 16)]
            return acc_loss + _log_sigmoid(x)

        loss = lax.fori_loop(0, _NGRP, group_body, loss)
        if c + 2 < _NCHUNK:
            fire(c + 2)

    zeros = jnp.zeros((16,), jnp.float32)
    for k in range(8):
        stage_v[pl.ds(k * 16, 16)] = loss if k == 0 else zeros
    pltpu.sync_copy(stage_v, out_hbm.at[wid])


def kernel(u, v, sign, emb, ctx):
    emb2 = emb.reshape(NUM_NODES // 2, 2 * DIM)
    ctx2 = ctx.reshape(NUM_NODES // 2, 2 * DIM)
    partials = _sc_loss(u.astype(jnp.int32), v.astype(jnp.int32),
                        sign, emb2, ctx2)
    return -jnp.sum(partials)
